# baseline (speedup 1.0000x reference)
import jax
import jax.numpy as jnp
from jax.experimental import pallas as pl
from jax.experimental.pallas import tpu as pltpu


def _dma_kernel(x_ref, o_ref, sem):
    n = x_ref.shape[0]
    for i in range(n):
        pltpu.make_async_copy(x_ref.at[i], o_ref.at[n - 1 - i], sem).start()
    for i in range(n):
        pltpu.make_async_copy(x_ref.at[i], o_ref.at[n - 1 - i], sem).wait()


def kernel(x):
    return pl.pallas_call(
        _dma_kernel,
        in_specs=[pl.BlockSpec(memory_space=pl.ANY)],
        out_specs=pl.BlockSpec(memory_space=pl.ANY),
        out_shape=jax.ShapeDtypeStruct(x.shape, x.dtype),
        scratch_shapes=[pltpu.SemaphoreType.DMA],
    )(x)


# speedup vs baseline: 42.7324x; 42.7324x over previous
import jax
import jax.numpy as jnp
from jax.experimental import pallas as pl
from jax.experimental.pallas import tpu as pltpu


def _copy_kernel(x_ref, o_ref):
    o_ref[...] = x_ref[...]


def kernel(x):
    n, r, c = x.shape
    BR = 512
    grid = (n, r // BR)
    return pl.pallas_call(
        _copy_kernel,
        grid=grid,
        in_specs=[pl.BlockSpec((1, BR, c), lambda i, j: (n - 1 - i, j, 0))],
        out_specs=pl.BlockSpec((1, BR, c), lambda i, j: (i, j, 0)),
        out_shape=jax.ShapeDtypeStruct((n, r, c), x.dtype),
        compiler_params=pltpu.CompilerParams(
            dimension_semantics=("parallel", "parallel"),
        ),
    )(x)


# speedup vs baseline: 46.3483x; 1.0846x over previous
import jax
import jax.numpy as jnp
from jax.experimental import pallas as pl
from jax.experimental.pallas import tpu as pltpu


def _copy_kernel(x_ref, o_ref):
    o_ref[...] = x_ref[...]


def kernel(x):
    n, r, c = x.shape
    BR = 1024
    grid = (n, r // BR)
    return pl.pallas_call(
        _copy_kernel,
        grid=grid,
        in_specs=[pl.BlockSpec((1, BR, c), lambda i, j: (n - 1 - i, j, 0))],
        out_specs=pl.BlockSpec((1, BR, c), lambda i, j: (i, j, 0)),
        out_shape=jax.ShapeDtypeStruct((n, r, c), x.dtype),
        compiler_params=pltpu.CompilerParams(
            dimension_semantics=("parallel", "parallel"),
        ),
    )(x)


# speedup vs baseline: 47.9540x; 1.0346x over previous
import jax
import jax.numpy as jnp
from jax.experimental import pallas as pl
from jax.experimental.pallas import tpu as pltpu


def _copy_kernel(x_ref, o_ref):
    o_ref[...] = x_ref[...]


def kernel(x):
    n, r, c = x.shape
    BR = 2048
    grid = (n, r // BR)
    return pl.pallas_call(
        _copy_kernel,
        grid=grid,
        in_specs=[pl.BlockSpec((1, BR, c), lambda i, j: (n - 1 - i, j, 0))],
        out_specs=pl.BlockSpec((1, BR, c), lambda i, j: (i, j, 0)),
        out_shape=jax.ShapeDtypeStruct((n, r, c), x.dtype),
        compiler_params=pltpu.CompilerParams(
            dimension_semantics=("parallel", "parallel"),
        ),
    )(x)


# speedup vs baseline: 48.0070x; 1.0011x over previous
import jax
import jax.numpy as jnp
from jax.experimental import pallas as pl
from jax.experimental.pallas import tpu as pltpu


def _copy_kernel(x_ref, o_ref):
    o_ref[...] = x_ref[...]


def kernel(x):
    n, r, c = x.shape
    return pl.pallas_call(
        _copy_kernel,
        grid=(n,),
        in_specs=[pl.BlockSpec((1, r, c), lambda i: (n - 1 - i, 0, 0))],
        out_specs=pl.BlockSpec((1, r, c), lambda i: (i, 0, 0)),
        out_shape=jax.ShapeDtypeStruct((n, r, c), x.dtype),
        compiler_params=pltpu.CompilerParams(
            dimension_semantics=("parallel",),
        ),
    )(x)
